# Initial kernel scaffold; baseline (speedup 1.0000x reference)
import jax
import jax.numpy as jnp
from jax.experimental import pallas as pl


def kernel(x, edge_index, edge_weight, W1, a_src1, a_dst1, b1, W2, a_src2, a_dst2, b2):
    raise NotImplementedError("write your pallas kernel here")


# speedup vs baseline: 27.4922x; 27.4922x over previous
import functools

import jax
import jax.numpy as jnp
from jax import lax
from jax.experimental import pallas as pl
from jax.experimental.pallas import tpu as pltpu
from jax.experimental.pallas import tpu_sc as plsc

N = 10000
E = 320000
NC = 2
NS = 16
NW = NC * NS
EPW = E // NW
BB = 80
NB = EPW // BB
RP = 624
TAIL = N - NS * RP

f32 = jnp.float32


def _tc_pre_body(x_ref, w_ref, asf_ref, adf_ref, g_ref, xh_ref, as_ref, ad_ref):
    xb = x_ref[...]
    xh = jnp.dot(xb, w_ref[...], preferred_element_type=f32)
    xh_ref[...] = xh
    as_ref[...] = jnp.dot(xh * asf_ref[...], g_ref[...], preferred_element_type=f32)
    ad_ref[...] = jnp.dot(xh * adf_ref[...], g_ref[...], preferred_element_type=f32)


def _tc_pre(x, w, asf, adf, g):
    bn = 1000
    grid = (N // bn,)
    return pl.pallas_call(
        _tc_pre_body,
        grid=grid,
        in_specs=[
            pl.BlockSpec((bn, 128), lambda i: (i, 0)),
            pl.BlockSpec((128, 128), lambda i: (0, 0)),
            pl.BlockSpec((1, 128), lambda i: (0, 0)),
            pl.BlockSpec((1, 128), lambda i: (0, 0)),
            pl.BlockSpec((128, 8), lambda i: (0, 0)),
        ],
        out_specs=[
            pl.BlockSpec((bn, 128), lambda i: (i, 0)),
            pl.BlockSpec((bn, 8), lambda i: (i, 0)),
            pl.BlockSpec((bn, 8), lambda i: (i, 0)),
        ],
        out_shape=[
            jax.ShapeDtypeStruct((N, 128), f32),
            jax.ShapeDtypeStruct((N, 8), f32),
            jax.ShapeDtypeStruct((N, 8), f32),
        ],
    )(x, w, asf, adf, g)


def _tc_mid_body(p_ref, b_ref, w_ref, asf_ref, adf_ref, g_ref,
                 xh_ref, as_ref, ad_ref):
    p = p_ref[0] + p_ref[1] + b_ref[...]
    h = jnp.where(p > 0, p, jnp.exp(jnp.minimum(p, 0.0)) - 1.0)
    xh = jnp.dot(h, w_ref[...], preferred_element_type=f32)
    xh_ref[...] = xh
    as_ref[...] = jnp.dot(xh * asf_ref[...], g_ref[...], preferred_element_type=f32)
    ad_ref[...] = jnp.dot(xh * adf_ref[...], g_ref[...], preferred_element_type=f32)


def _tc_mid(op1, b, w, asf, adf, g):
    bn = 1000
    grid = (N // bn,)
    return pl.pallas_call(
        _tc_mid_body,
        grid=grid,
        in_specs=[
            pl.BlockSpec((2, bn, 128), lambda i: (0, i, 0)),
            pl.BlockSpec((1, 128), lambda i: (0, 0)),
            pl.BlockSpec((128, 128), lambda i: (0, 0)),
            pl.BlockSpec((1, 128), lambda i: (0, 0)),
            pl.BlockSpec((1, 128), lambda i: (0, 0)),
            pl.BlockSpec((128, 8), lambda i: (0, 0)),
        ],
        out_specs=[
            pl.BlockSpec((bn, 128), lambda i: (i, 0)),
            pl.BlockSpec((bn, 8), lambda i: (i, 0)),
            pl.BlockSpec((bn, 8), lambda i: (i, 0)),
        ],
        out_shape=[
            jax.ShapeDtypeStruct((N, 128), f32),
            jax.ShapeDtypeStruct((N, 8), f32),
            jax.ShapeDtypeStruct((N, 8), f32),
        ],
    )(op1, b, w, asf, adf, g)


def _tc_tables_body(as_ref, ad_ref, st_ref, dt_ref, ct_ref):
    asrc = as_ref[...]
    adst = ad_ref[...]
    amax = jnp.max(asrc, axis=0, keepdims=True)
    t = adst + amax
    c = jnp.maximum(t, 0.2 * t)
    z = jnp.zeros_like(asrc)
    st_ref[...] = jnp.concatenate([asrc, z], axis=1)
    dt_ref[...] = jnp.concatenate([adst, z], axis=1)
    ct_ref[...] = jnp.concatenate([c, z], axis=1)


def _tc_tables(asrc, adst):
    return pl.pallas_call(
        _tc_tables_body,
        out_shape=[
            jax.ShapeDtypeStruct((N, 16), f32),
            jax.ShapeDtypeStruct((N, 16), f32),
            jax.ShapeDtypeStruct((N, 16), f32),
        ],
    )(asrc, adst)


def _tc_recip_body(sp_ref, rt_ref):
    rt_ref[...] = 1.0 / (sp_ref[0] + sp_ref[1] + 1e-16)


def _tc_recip(sp):
    bn = 2000
    grid = (N // bn,)
    return pl.pallas_call(
        _tc_recip_body,
        grid=grid,
        in_specs=[pl.BlockSpec((2, bn, 16), lambda i: (0, i, 0))],
        out_specs=pl.BlockSpec((bn, 16), lambda i: (i, 0)),
        out_shape=jax.ShapeDtypeStruct((N, 16), f32),
    )(sp)


def _tc_fin_body(op_ref, b_ref, o_ref):
    o_ref[...] = op_ref[0] + op_ref[1] + b_ref[...]


def _tc_fin(op2, b):
    bn = 2000
    grid = (N // bn,)
    return pl.pallas_call(
        _tc_fin_body,
        grid=grid,
        in_specs=[
            pl.BlockSpec((2, bn, 128), lambda i: (0, i, 0)),
            pl.BlockSpec((1, 128), lambda i: (0, 0)),
        ],
        out_specs=pl.BlockSpec((bn, 128), lambda i: (i, 0)),
        out_shape=jax.ShapeDtypeStruct((N, 128), f32),
    )(op2, b)


_MESH = plsc.VectorSubcoreMesh(core_axis_name="c", subcore_axis_name="s")
_SC_PARAMS = pltpu.CompilerParams(use_tc_tiling_on_sc=False)


def _sc_pass_a_body(src_ref, dst_ref, st_ref, dt_ref, ct_ref, z_ref,
                    ev_ref, sp_ref,
                    idxs, idxd, ra, rd, rc, eb, s_sh):
    cid = lax.axis_index("c")
    sid = lax.axis_index("s")
    wid = sid * NC + cid
    pltpu.sync_copy(z_ref.at[pl.ds(sid * RP, RP)], s_sh.at[pl.ds(sid * RP, RP)])

    @pl.when(sid == 0)
    def _():
        pltpu.sync_copy(z_ref.at[pl.ds(NS * RP, TAIL)],
                        s_sh.at[pl.ds(NS * RP, TAIL)])

    plsc.subcore_barrier()
    pltpu.sync_copy(src_ref.at[wid], idxs)
    pltpu.sync_copy(dst_ref.at[wid], idxd)

    def blk(j, carry):
        pltpu.sync_copy(st_ref.at[idxs.at[j]], ra)
        pltpu.sync_copy(dt_ref.at[idxd.at[j]], rd)
        pltpu.sync_copy(ct_ref.at[idxd.at[j]], rc)

        def row(i, c2):
            t = ra[i, :] + rd[i, :]
            l = jnp.maximum(t, 0.2 * t)
            eb[i, :] = jnp.exp(l - rc[i, :])
            return c2

        lax.fori_loop(0, BB, row, 0)
        pltpu.sync_copy(eb, ev_ref.at[wid, j])
        pltpu.sync_copy(eb, s_sh.at[idxd.at[j]], add=True)
        return carry

    lax.fori_loop(0, NB, blk, 0)
    plsc.subcore_barrier()
    pltpu.sync_copy(s_sh.at[pl.ds(sid * RP, RP)],
                    sp_ref.at[cid, pl.ds(sid * RP, RP)])

    @pl.when(sid == 0)
    def _():
        pltpu.sync_copy(s_sh.at[pl.ds(NS * RP, TAIL)],
                        sp_ref.at[cid, pl.ds(NS * RP, TAIL)])


_sc_pass_a = pl.kernel(
    _sc_pass_a_body,
    out_type=[
        jax.ShapeDtypeStruct((NW, NB, BB, 16), f32),
        jax.ShapeDtypeStruct((NC, N, 16), f32),
    ],
    mesh=_MESH,
    compiler_params=_SC_PARAMS,
    scratch_types=[
        pltpu.VMEM((NB, BB), jnp.int32),
        pltpu.VMEM((NB, BB), jnp.int32),
        pltpu.VMEM((BB, 16), f32),
        pltpu.VMEM((BB, 16), f32),
        pltpu.VMEM((BB, 16), f32),
        pltpu.VMEM((BB, 16), f32),
        pltpu.VMEM_SHARED((N, 16), f32),
    ],
)


def _sc_pass_b_body(nheads, src_ref, dst_ref, xh_ref, rt_ref, ev_ref, z_ref,
                    op_ref,
                    idxs, idxd, xb, rb, ebuf, o_sh):
    cid = lax.axis_index("c")
    sid = lax.axis_index("s")
    wid = sid * NC + cid
    pltpu.sync_copy(z_ref.at[pl.ds(sid * RP, RP)], o_sh.at[pl.ds(sid * RP, RP)])

    @pl.when(sid == 0)
    def _():
        pltpu.sync_copy(z_ref.at[pl.ds(NS * RP, TAIL)],
                        o_sh.at[pl.ds(NS * RP, TAIL)])

    plsc.subcore_barrier()
    pltpu.sync_copy(src_ref.at[wid], idxs)
    pltpu.sync_copy(dst_ref.at[wid], idxd)

    def blk(j, carry):
        pltpu.sync_copy(xh_ref.at[idxs.at[j]], xb)
        pltpu.sync_copy(rt_ref.at[idxd.at[j]], rb)
        pltpu.sync_copy(ev_ref.at[wid, j], ebuf)

        def row(i, c2):
            w16 = ebuf[i, :] * rb[i, :]
            for k in range(8):
                wk = w16[k] if nheads == 8 else w16[0]
                sl = pl.ds(k * 16, 16)
                xb[i, sl] = xb[i, sl] * wk
            return c2

        lax.fori_loop(0, BB, row, 0)
        pltpu.sync_copy(xb, o_sh.at[idxd.at[j]], add=True)
        return carry

    lax.fori_loop(0, NB, blk, 0)
    plsc.subcore_barrier()
    pltpu.sync_copy(o_sh.at[pl.ds(sid * RP, RP)],
                    op_ref.at[cid, pl.ds(sid * RP, RP)])

    @pl.when(sid == 0)
    def _():
        pltpu.sync_copy(o_sh.at[pl.ds(NS * RP, TAIL)],
                        op_ref.at[cid, pl.ds(NS * RP, TAIL)])


def _make_sc_pass_b(nheads):
    return pl.kernel(
        functools.partial(_sc_pass_b_body, nheads),
        out_type=jax.ShapeDtypeStruct((NC, N, 128), f32),
        mesh=_MESH,
        compiler_params=_SC_PARAMS,
        scratch_types=[
            pltpu.VMEM((NB, BB), jnp.int32),
            pltpu.VMEM((NB, BB), jnp.int32),
            pltpu.VMEM((BB, 128), f32),
            pltpu.VMEM((BB, 16), f32),
            pltpu.VMEM((BB, 16), f32),
            pltpu.VMEM_SHARED((N, 128), f32),
        ],
    )


_sc_pass_b8 = _make_sc_pass_b(8)
_sc_pass_b1 = _make_sc_pass_b(1)


def kernel(x, edge_index, edge_weight, W1, a_src1, a_dst1, b1,
           W2, a_src2, a_dst2, b2):
    del edge_weight
    src_r = edge_index[0].reshape(NW, NB, BB)
    dst_r = edge_index[1].reshape(NW, NB, BB)
    z16 = jnp.zeros((N, 16), f32)
    z128 = jnp.zeros((N, 128), f32)
    g8 = jnp.repeat(jnp.eye(8, dtype=f32), 16, axis=0)
    g1 = jnp.zeros((128, 8), f32).at[:, 0].set(1.0)
    asf1 = a_src1.reshape(1, 128)
    adf1 = a_dst1.reshape(1, 128)
    asf2 = a_src2.reshape(1, 128)
    adf2 = a_dst2.reshape(1, 128)

    xh1, as1, ad1 = _tc_pre(x, W1, asf1, adf1, g8)
    st1, dt1, ct1 = _tc_tables(as1, ad1)
    ev1, sp1 = _sc_pass_a(src_r, dst_r, st1, dt1, ct1, z16)
    rt1 = _tc_recip(sp1)
    op1 = _sc_pass_b8(src_r, dst_r, xh1, rt1, ev1, z128)

    xh2, as2, ad2 = _tc_mid(op1, b1.reshape(1, 128), W2, asf2, adf2, g1)
    st2, dt2, ct2 = _tc_tables(as2, ad2)
    ev2, sp2 = _sc_pass_a(src_r, dst_r, st2, dt2, ct2, z16)
    rt2 = _tc_recip(sp2)
    op2 = _sc_pass_b1(src_r, dst_r, xh2, rt2, ev2, z128)

    return _tc_fin(op2, b2.reshape(1, 128))


# speedup vs baseline: 60.1815x; 2.1890x over previous
import jax
import jax.numpy as jnp
from jax import lax
from jax.experimental import pallas as pl
from jax.experimental.pallas import tpu as pltpu
from jax.experimental.pallas import tpu_sc as plsc

N = 10000
E = 320000
NC = 2
NS = 16
NW = NC * NS
EPW = E // NW
BB = 80
NB = EPW // BB
RP = 624
TAIL = N - NS * RP

f32 = jnp.float32


def _tc_pre_body(x_ref, w_ref, asf_ref, adf_ref, g_ref, xh_ref, as_ref, ad_ref):
    xb = x_ref[...]
    xh = jnp.dot(xb, w_ref[...], preferred_element_type=f32)
    xh_ref[...] = xh
    as_ref[...] = jnp.dot(xh * asf_ref[...], g_ref[...], preferred_element_type=f32)
    ad_ref[...] = jnp.dot(xh * adf_ref[...], g_ref[...], preferred_element_type=f32)


def _tc_pre(x, w, asf, adf, g):
    bn = 1000
    grid = (N // bn,)
    return pl.pallas_call(
        _tc_pre_body,
        grid=grid,
        in_specs=[
            pl.BlockSpec((bn, 128), lambda i: (i, 0)),
            pl.BlockSpec((128, 128), lambda i: (0, 0)),
            pl.BlockSpec((1, 128), lambda i: (0, 0)),
            pl.BlockSpec((1, 128), lambda i: (0, 0)),
            pl.BlockSpec((128, 8), lambda i: (0, 0)),
        ],
        out_specs=[
            pl.BlockSpec((bn, 128), lambda i: (i, 0)),
            pl.BlockSpec((bn, 8), lambda i: (i, 0)),
            pl.BlockSpec((bn, 8), lambda i: (i, 0)),
        ],
        out_shape=[
            jax.ShapeDtypeStruct((N, 128), f32),
            jax.ShapeDtypeStruct((N, 8), f32),
            jax.ShapeDtypeStruct((N, 8), f32),
        ],
    )(x, w, asf, adf, g)


def _tc_mid_body(op_ref, sp_ref, b_ref, w_ref, asf_ref, adf_ref, g_ref, r_ref,
                 xh_ref, as_ref, ad_ref):
    s8 = sp_ref[0, :, 0:8] + sp_ref[1, :, 0:8]
    srep = jnp.dot(s8, r_ref[...], preferred_element_type=f32)
    p = (op_ref[0] + op_ref[1]) / (srep + 1e-16) + b_ref[...]
    h = jnp.where(p > 0, p, jnp.exp(jnp.minimum(p, 0.0)) - 1.0)
    xh = jnp.dot(h, w_ref[...], preferred_element_type=f32)
    xh_ref[...] = xh
    as_ref[...] = jnp.dot(xh * asf_ref[...], g_ref[...], preferred_element_type=f32)
    ad_ref[...] = jnp.dot(xh * adf_ref[...], g_ref[...], preferred_element_type=f32)


def _tc_mid(op1, sp1, b, w, asf, adf, g, r):
    bn = 1000
    grid = (N // bn,)
    return pl.pallas_call(
        _tc_mid_body,
        grid=grid,
        in_specs=[
            pl.BlockSpec((2, bn, 128), lambda i: (0, i, 0)),
            pl.BlockSpec((2, bn, 16), lambda i: (0, i, 0)),
            pl.BlockSpec((1, 128), lambda i: (0, 0)),
            pl.BlockSpec((128, 128), lambda i: (0, 0)),
            pl.BlockSpec((1, 128), lambda i: (0, 0)),
            pl.BlockSpec((1, 128), lambda i: (0, 0)),
            pl.BlockSpec((128, 8), lambda i: (0, 0)),
            pl.BlockSpec((8, 128), lambda i: (0, 0)),
        ],
        out_specs=[
            pl.BlockSpec((bn, 128), lambda i: (i, 0)),
            pl.BlockSpec((bn, 8), lambda i: (i, 0)),
            pl.BlockSpec((bn, 8), lambda i: (i, 0)),
        ],
        out_shape=[
            jax.ShapeDtypeStruct((N, 128), f32),
            jax.ShapeDtypeStruct((N, 8), f32),
            jax.ShapeDtypeStruct((N, 8), f32),
        ],
    )(op1, sp1, b, w, asf, adf, g, r)


def _tc_tables_body(as_ref, ad_ref, st_ref, dt_ref, ct_ref):
    asrc = as_ref[...]
    adst = ad_ref[...]
    amax = jnp.max(asrc, axis=0, keepdims=True)
    t = adst + amax
    c = jnp.maximum(t, 0.2 * t)
    z = jnp.zeros_like(asrc)
    st_ref[...] = jnp.concatenate([asrc, z], axis=1)
    dt_ref[...] = jnp.concatenate([adst, z], axis=1)
    ct_ref[...] = jnp.concatenate([c, z], axis=1)


def _tc_tables(asrc, adst):
    return pl.pallas_call(
        _tc_tables_body,
        out_shape=[
            jax.ShapeDtypeStruct((N, 16), f32),
            jax.ShapeDtypeStruct((N, 16), f32),
            jax.ShapeDtypeStruct((N, 16), f32),
        ],
    )(asrc, adst)


def _tc_fin_body(op_ref, sp_ref, b_ref, o_ref):
    s0 = sp_ref[0, :, 0:1] + sp_ref[1, :, 0:1]
    o_ref[...] = (op_ref[0] + op_ref[1]) / (s0 + 1e-16) + b_ref[...]


def _tc_fin(op2, sp2, b):
    bn = 2000
    grid = (N // bn,)
    return pl.pallas_call(
        _tc_fin_body,
        grid=grid,
        in_specs=[
            pl.BlockSpec((2, bn, 128), lambda i: (0, i, 0)),
            pl.BlockSpec((2, bn, 16), lambda i: (0, i, 0)),
            pl.BlockSpec((1, 128), lambda i: (0, 0)),
        ],
        out_specs=pl.BlockSpec((bn, 128), lambda i: (i, 0)),
        out_shape=jax.ShapeDtypeStruct((N, 128), f32),
    )(op2, sp2, b)


_MESH = plsc.VectorSubcoreMesh(core_axis_name="c", subcore_axis_name="s")
_SC_PARAMS = pltpu.CompilerParams(use_tc_tiling_on_sc=False)


def _sc_pass_a_body(src_ref, dst_ref, st_ref, dt_ref, ct_ref, z_ref,
                    ev_ref, sp_ref,
                    idxs, idxd, ra, rd, rc, eb, sem0, sem1, s_sh):
    cid = lax.axis_index("c")
    sid = lax.axis_index("s")
    wid = sid * NC + cid
    sems = (sem0, sem1)
    pltpu.sync_copy(z_ref.at[pl.ds(sid * RP, RP)], s_sh.at[pl.ds(sid * RP, RP)])

    @pl.when(sid == 0)
    def _():
        pltpu.sync_copy(z_ref.at[pl.ds(NS * RP, TAIL)],
                        s_sh.at[pl.ds(NS * RP, TAIL)])

    plsc.subcore_barrier()
    pltpu.sync_copy(src_ref.at[wid], idxs)
    pltpu.sync_copy(dst_ref.at[wid], idxd)

    def start(j, b):
        pltpu.async_copy(st_ref.at[idxs.at[j]], ra.at[b], sems[b])
        pltpu.async_copy(dt_ref.at[idxd.at[j]], rd.at[b], sems[b])
        pltpu.async_copy(ct_ref.at[idxd.at[j]], rc.at[b], sems[b])

    def wait(j, b):
        pltpu.make_async_copy(st_ref.at[idxs.at[j]], ra.at[b], sems[b]).wait()
        pltpu.make_async_copy(dt_ref.at[idxd.at[j]], rd.at[b], sems[b]).wait()
        pltpu.make_async_copy(ct_ref.at[idxd.at[j]], rc.at[b], sems[b]).wait()

    def work(j, b):
        def row(i, c2):
            t = ra[b, i, :] + rd[b, i, :]
            l = jnp.maximum(t, 0.2 * t)
            eb[b, i, :] = jnp.exp(l - rc[b, i, :])
            return c2

        lax.fori_loop(0, BB, row, 0, unroll=4)
        pltpu.sync_copy(eb.at[b], ev_ref.at[wid, j])
        pltpu.sync_copy(eb.at[b], s_sh.at[idxd.at[j]], add=True)

    start(0, 0)

    def pair(i, carry):
        j0 = 2 * i
        start(j0 + 1, 1)
        wait(j0, 0)
        work(j0, 0)
        start(j0 + 2, 0)
        wait(j0 + 1, 1)
        work(j0 + 1, 1)
        return carry

    lax.fori_loop(0, (NB - 1) // 2, pair, 0)
    wait(NB - 1, 0)
    work(NB - 1, 0)
    plsc.subcore_barrier()
    pltpu.sync_copy(s_sh.at[pl.ds(sid * RP, RP)],
                    sp_ref.at[cid, pl.ds(sid * RP, RP)])

    @pl.when(sid == 0)
    def _():
        pltpu.sync_copy(s_sh.at[pl.ds(NS * RP, TAIL)],
                        sp_ref.at[cid, pl.ds(NS * RP, TAIL)])


_sc_pass_a = pl.kernel(
    _sc_pass_a_body,
    out_type=[
        jax.ShapeDtypeStruct((NW, NB, BB, 16), f32),
        jax.ShapeDtypeStruct((NC, N, 16), f32),
    ],
    mesh=_MESH,
    compiler_params=_SC_PARAMS,
    scratch_types=[
        pltpu.VMEM((NB, BB), jnp.int32),
        pltpu.VMEM((NB, BB), jnp.int32),
        pltpu.VMEM((2, BB, 16), f32),
        pltpu.VMEM((2, BB, 16), f32),
        pltpu.VMEM((2, BB, 16), f32),
        pltpu.VMEM((2, BB, 16), f32),
        pltpu.SemaphoreType.DMA,
        pltpu.SemaphoreType.DMA,
        pltpu.VMEM_SHARED((N, 16), f32),
    ],
)


def _sc_pass_b_body(nheads, src_ref, dst_ref, xh_ref, ev_ref, z_ref,
                    op_ref,
                    idxs, idxd, xb, ebuf, sem0, sem1, o_sh):
    cid = lax.axis_index("c")
    sid = lax.axis_index("s")
    wid = sid * NC + cid
    sems = (sem0, sem1)
    pltpu.sync_copy(z_ref.at[pl.ds(sid * RP, RP)], o_sh.at[pl.ds(sid * RP, RP)])

    @pl.when(sid == 0)
    def _():
        pltpu.sync_copy(z_ref.at[pl.ds(NS * RP, TAIL)],
                        o_sh.at[pl.ds(NS * RP, TAIL)])

    plsc.subcore_barrier()
    pltpu.sync_copy(src_ref.at[wid], idxs)
    pltpu.sync_copy(dst_ref.at[wid], idxd)

    def start(j, b):
        pltpu.async_copy(xh_ref.at[idxs.at[j]], xb.at[b], sems[b])
        pltpu.async_copy(ev_ref.at[wid, j], ebuf.at[b], sems[b])

    def wait(j, b):
        pltpu.make_async_copy(xh_ref.at[idxs.at[j]], xb.at[b], sems[b]).wait()
        pltpu.make_async_copy(ev_ref.at[wid, j], ebuf.at[b], sems[b]).wait()

    def work(j, b):
        def row(i, c2):
            w16 = ebuf[b, i, :]
            for k in range(8):
                wk = w16[k] if nheads == 8 else w16[0]
                sl = pl.ds(k * 16, 16)
                xb[b, i, sl] = xb[b, i, sl] * wk
            return c2

        lax.fori_loop(0, BB, row, 0, unroll=2)
        pltpu.sync_copy(xb.at[b], o_sh.at[idxd.at[j]], add=True)

    start(0, 0)

    def pair(i, carry):
        j0 = 2 * i
        start(j0 + 1, 1)
        wait(j0, 0)
        work(j0, 0)
        start(j0 + 2, 0)
        wait(j0 + 1, 1)
        work(j0 + 1, 1)
        return carry

    lax.fori_loop(0, (NB - 1) // 2, pair, 0)
    wait(NB - 1, 0)
    work(NB - 1, 0)
    plsc.subcore_barrier()
    pltpu.sync_copy(o_sh.at[pl.ds(sid * RP, RP)],
                    op_ref.at[cid, pl.ds(sid * RP, RP)])

    @pl.when(sid == 0)
    def _():
        pltpu.sync_copy(o_sh.at[pl.ds(NS * RP, TAIL)],
                        op_ref.at[cid, pl.ds(NS * RP, TAIL)])


def _make_sc_pass_b(nheads):
    def body(*args):
        _sc_pass_b_body(nheads, *args)

    return pl.kernel(
        body,
        out_type=jax.ShapeDtypeStruct((NC, N, 128), f32),
        mesh=_MESH,
        compiler_params=_SC_PARAMS,
        scratch_types=[
            pltpu.VMEM((NB, BB), jnp.int32),
            pltpu.VMEM((NB, BB), jnp.int32),
            pltpu.VMEM((2, BB, 128), f32),
            pltpu.VMEM((2, BB, 16), f32),
            pltpu.SemaphoreType.DMA,
            pltpu.SemaphoreType.DMA,
            pltpu.VMEM_SHARED((N, 128), f32),
        ],
    )


_sc_pass_b8 = _make_sc_pass_b(8)
_sc_pass_b1 = _make_sc_pass_b(1)


def kernel(x, edge_index, edge_weight, W1, a_src1, a_dst1, b1,
           W2, a_src2, a_dst2, b2):
    del edge_weight
    src_r = edge_index[0].reshape(NW, NB, BB)
    dst_r = edge_index[1].reshape(NW, NB, BB)
    z16 = jnp.zeros((N, 16), f32)
    z128 = jnp.zeros((N, 128), f32)
    g8 = jnp.repeat(jnp.eye(8, dtype=f32), 16, axis=0)
    r8 = g8.T
    g1 = jnp.zeros((128, 8), f32).at[:, 0].set(1.0)
    asf1 = a_src1.reshape(1, 128)
    adf1 = a_dst1.reshape(1, 128)
    asf2 = a_src2.reshape(1, 128)
    adf2 = a_dst2.reshape(1, 128)

    xh1, as1, ad1 = _tc_pre(x, W1, asf1, adf1, g8)
    st1, dt1, ct1 = _tc_tables(as1, ad1)
    ev1, sp1 = _sc_pass_a(src_r, dst_r, st1, dt1, ct1, z16)
    op1 = _sc_pass_b8(src_r, dst_r, xh1, ev1, z128)

    xh2, as2, ad2 = _tc_mid(op1, sp1, b1.reshape(1, 128), W2, asf2, adf2,
                            g1, r8)
    st2, dt2, ct2 = _tc_tables(as2, ad2)
    ev2, sp2 = _sc_pass_a(src_r, dst_r, st2, dt2, ct2, z16)
    op2 = _sc_pass_b1(src_r, dst_r, xh2, ev2, z128)

    return _tc_fin(op2, sp2, b2.reshape(1, 128))
